# Initial kernel scaffold; baseline (speedup 1.0000x reference)
import jax
import jax.numpy as jnp
from jax.experimental import pallas as pl


def kernel(step, lprobs, scores, prev_output_tokens, original_batch_idxs, prefix_mention_is):
    raise NotImplementedError("write your pallas kernel here")


# speedup vs baseline: 14.4487x; 14.4487x over previous
import functools

import jax
import jax.numpy as jnp
from jax import lax
from jax.experimental import pallas as pl
from jax.experimental.pallas import tpu as pltpu
from jax.experimental.pallas import tpu_sc as plsc

_MULT = 977
_NALLOW = 100
_ROW_PAD = 112
_WIN = 2 * _ROW_PAD
_NEG = -3.4028235e38
_IMAX = 2**31 - 1


def _sc_body(vocab, nc, beam, lp_ref, last_ref, sc_ref, bidx_ref,
             scores_out, tokens_out, beams_out,
             bidx_v, last_v, sc_v, win_v, vals_v, cidx_v,
             ov_f, ot_i, ob_i, sem):
    nchunk = _ROW_PAD // 16
    ncc = beam * nchunk
    w = lax.axis_index("s") * nc + lax.axis_index("c")
    lane = jnp.arange(16, dtype=jnp.int32)

    pltpu.sync_copy(bidx_ref, bidx_v.at[pl.ds(0, bidx_ref.shape[0])])
    pltpu.sync_copy(last_ref, last_v.at[pl.ds(0, last_ref.shape[0])])
    pltpu.sync_copy(sc_ref, sc_v.at[pl.ds(0, sc_ref.shape[0])])
    b_id = bidx_v[pl.ds(w, 16)][0]

    handles = []
    row_info = []
    for m in range(beam):
        r = w * beam + m
        base = lax.rem(b_id * _MULT + last_v[pl.ds(r, 16)][0], vocab)
        s1 = lax.min(base - lax.rem(base, 8), vocab - _ROW_PAD)
        row_info.append((base, s1))
        handles.append(pltpu.async_copy(
            lp_ref.at[pl.ds(pl.multiple_of(r * vocab + s1, 8), _ROW_PAD)],
            win_v.at[pl.ds(m * _WIN, _ROW_PAD)], sem))
        handles.append(pltpu.async_copy(
            lp_ref.at[pl.ds(pl.multiple_of(r * vocab, 8), _ROW_PAD)],
            win_v.at[pl.ds(m * _WIN + _ROW_PAD, _ROW_PAD)], sem))
    for h in handles:
        h.wait()

    for m in range(beam):
        base, s1 = row_info[m]
        sc_m = sc_v[pl.ds(w * beam + m, 16)][0]
        for c in range(nchunk):
            j = lane + 16 * c
            idv = base + j
            wrapped = idv >= vocab
            tok = jnp.where(wrapped, idv - vocab, idv)
            off = jnp.where(wrapped, tok + _ROW_PAD, idv - s1)
            valid = j < _NALLOW
            off = jnp.where(valid, off + m * _WIN, 0)
            g = plsc.load_gather(win_v, [off])
            vals_v[pl.ds(m * _ROW_PAD + 16 * c, 16)] = jnp.where(valid, g + sc_m, _NEG)
            cidx_v[pl.ds(m * _ROW_PAD + 16 * c, 16)] = jnp.where(valid, m * vocab + tok, _IMAX)

    out_val = jnp.full((16,), _NEG, jnp.float32)
    out_idx = jnp.full((16,), _IMAX, jnp.int32)

    def scan_body(cc, carry):
        bv, bi, bp = carry
        v = vals_v[pl.ds(cc * 16, 16)]
        i = cidx_v[pl.ds(cc * 16, 16)]
        p = lane + cc * 16
        upd = (v > bv) | ((v == bv) & (i < bi))
        return (jnp.where(upd, v, bv), jnp.where(upd, i, bi), jnp.where(upd, p, bp))

    for rnd in range(8):
        init = (jnp.full((16,), _NEG, jnp.float32),
                jnp.full((16,), _IMAX, jnp.int32),
                jnp.full((16,), _IMAX, jnp.int32))
        bv, bi, bp = lax.fori_loop(0, ncc, scan_body, init)
        mval = jnp.max(bv)
        wi = jnp.min(jnp.where(bv == mval, bi, _IMAX))
        wp = jnp.min(jnp.where((bv == mval) & (bi == wi), bp, _IMAX))
        out_val = jnp.where(lane == rnd, mval, out_val)
        out_idx = jnp.where(lane == rnd, wi, out_idx)
        plsc.store_scatter(vals_v, [jnp.full((16,), wp, jnp.int32)],
                           jnp.full((16,), _NEG, jnp.float32), mask=lane == 0)

    beams = lax.div(out_idx, jnp.int32(vocab))
    toks = out_idx - beams * vocab
    ov_f[...] = out_val
    ot_i[...] = toks
    ob_i[...] = beams
    pltpu.sync_copy(ov_f, scores_out.at[pl.ds(w * 16, 16)])
    pltpu.sync_copy(ot_i, tokens_out.at[pl.ds(w * 16, 16)])
    pltpu.sync_copy(ob_i, beams_out.at[pl.ds(w * 16, 16)])


def kernel(step, lprobs, scores, prev_output_tokens, original_batch_idxs, prefix_mention_is):
    bsz, beam, vocab = lprobs.shape
    nrows = bsz * beam
    lp1 = lprobs.reshape(nrows * vocab)
    last = prev_output_tokens[:, -1].astype(jnp.int32)
    sc_sel = jnp.take(scores, step - 1, axis=2).reshape(nrows).astype(jnp.float32)
    bidx = original_batch_idxs.astype(jnp.int32)

    info = plsc.get_sparse_core_info()
    nc, ns = info.num_cores, info.num_subcores
    assert nc * ns == bsz, (nc, ns, bsz)

    mesh = plsc.VectorSubcoreMesh(core_axis_name="c", subcore_axis_name="s")
    f = pl.kernel(
        functools.partial(_sc_body, vocab, nc, beam),
        out_type=(
            jax.ShapeDtypeStruct((bsz * 16,), jnp.float32),
            jax.ShapeDtypeStruct((bsz * 16,), jnp.int32),
            jax.ShapeDtypeStruct((bsz * 16,), jnp.int32),
        ),
        mesh=mesh,
        compiler_params=pltpu.CompilerParams(needs_layout_passes=False),
        scratch_types=(
            pltpu.VMEM((bsz + 16,), jnp.int32),
            pltpu.VMEM((nrows + 16,), jnp.int32),
            pltpu.VMEM((nrows + 16,), jnp.float32),
            pltpu.VMEM((beam * _WIN,), jnp.float32),
            pltpu.VMEM((beam * _ROW_PAD,), jnp.float32),
            pltpu.VMEM((beam * _ROW_PAD,), jnp.int32),
            pltpu.VMEM((16,), jnp.float32),
            pltpu.VMEM((16,), jnp.int32),
            pltpu.VMEM((16,), jnp.int32),
            pltpu.SemaphoreType.DMA,
        ),
    )
    s_out, t_out, b_out = f(lp1, last, sc_sel, bidx)
    return (s_out.reshape(bsz, 16)[:, :beam],
            t_out.reshape(bsz, 16)[:, :beam],
            b_out.reshape(bsz, 16)[:, :beam])


# speedup vs baseline: 76.0433x; 5.2630x over previous
import functools

import jax
import jax.numpy as jnp
from jax import lax
from jax.experimental import pallas as pl
from jax.experimental.pallas import tpu as pltpu
from jax.experimental.pallas import tpu_sc as plsc

_MULT = 977
_NALLOW = 100
_ROW_PAD = 112
_WINW = 256
_NEG = -3.4028235e38
_IMAX = 2**31 - 1


def _sc_body(vocab, nc, beam, lp_ref, last_ref, sc_ref, bidx_ref,
             scores_out, tokens_out, beams_out,
             bidx_v, last_v, sc_v, win_v, vals_v, cidx_v,
             ov_f, ot_i, ob_i, sem):
    nchunk = _ROW_PAD // 16
    ncc = beam * nchunk
    w = lax.axis_index("s") * nc + lax.axis_index("c")
    lane = jnp.arange(16, dtype=jnp.int32)

    pltpu.sync_copy(bidx_ref, bidx_v.at[pl.ds(0, bidx_ref.shape[0])])
    pltpu.sync_copy(last_ref, last_v.at[pl.ds(0, last_ref.shape[0])])
    pltpu.sync_copy(sc_ref, sc_v.at[pl.ds(0, sc_ref.shape[0])])
    b_id = bidx_v[pl.ds(w, 16)][0]

    rows0 = pl.multiple_of(w * beam, 8)
    handles = [pltpu.async_copy(
        lp_ref.at[pl.ds(rows0, beam), pl.ds(0, _WINW)],
        win_v.at[beam], sem)]
    row_info = []
    for m in range(beam):
        base = lax.rem(b_id * _MULT + last_v[pl.ds(w * beam + m, 16)][0], vocab)
        s1 = lax.min(base - lax.rem(base, 128), ((vocab + 127) // 128) * 128 - _WINW)
        row_info.append((base, s1))
        handles.append(pltpu.async_copy(
            lp_ref.at[pl.ds(rows0, beam), pl.ds(pl.multiple_of(s1, 128), _WINW)],
            win_v.at[m], sem))
    for h in handles:
        h.wait()

    for m in range(beam):
        base, s1 = row_info[m]
        sc_m = sc_v[pl.ds(w * beam + m, 16)][0]
        row = jnp.full((16,), m, jnp.int32)
        for c in range(nchunk):
            j = lane + 16 * c
            idv = base + j
            wrapped = idv >= vocab
            tok = jnp.where(wrapped, idv - vocab, idv)
            sel = jnp.where(wrapped, beam, m)
            off = jnp.where(wrapped, tok, idv - s1)
            valid = j < _NALLOW
            off = jnp.where(valid, off, 0)
            g = plsc.load_gather(win_v, [sel, row, off])
            vals_v[pl.ds(m * _ROW_PAD + 16 * c, 16)] = jnp.where(valid, g + sc_m, _NEG)
            cidx_v[pl.ds(m * _ROW_PAD + 16 * c, 16)] = jnp.where(valid, m * vocab + tok, _IMAX)

    out_val = jnp.full((16,), _NEG, jnp.float32)
    out_idx = jnp.full((16,), _IMAX, jnp.int32)

    def scan_body(cc, carry):
        bv, bi, bp = carry
        v = vals_v[pl.ds(cc * 16, 16)]
        i = cidx_v[pl.ds(cc * 16, 16)]
        p = lane + cc * 16
        upd = (v > bv) | ((v == bv) & (i < bi))
        return (jnp.where(upd, v, bv), jnp.where(upd, i, bi), jnp.where(upd, p, bp))

    for rnd in range(8):
        init = (jnp.full((16,), _NEG, jnp.float32),
                jnp.full((16,), _IMAX, jnp.int32),
                jnp.full((16,), _IMAX, jnp.int32))
        bv, bi, bp = lax.fori_loop(0, ncc, scan_body, init)
        mval = jnp.max(bv)
        wi = jnp.min(jnp.where(bv == mval, bi, _IMAX))
        wp = jnp.min(jnp.where((bv == mval) & (bi == wi), bp, _IMAX))
        out_val = jnp.where(lane == rnd, mval, out_val)
        out_idx = jnp.where(lane == rnd, wi, out_idx)
        plsc.store_scatter(vals_v, [jnp.full((16,), wp, jnp.int32)],
                           jnp.full((16,), _NEG, jnp.float32), mask=lane == 0)

    beams = lax.div(out_idx, jnp.int32(vocab))
    toks = out_idx - beams * vocab
    ov_f[...] = out_val
    ot_i[...] = toks
    ob_i[...] = beams
    pltpu.sync_copy(ov_f, scores_out.at[pl.ds(w * 16, 16)])
    pltpu.sync_copy(ot_i, tokens_out.at[pl.ds(w * 16, 16)])
    pltpu.sync_copy(ob_i, beams_out.at[pl.ds(w * 16, 16)])


def kernel(step, lprobs, scores, prev_output_tokens, original_batch_idxs, prefix_mention_is):
    bsz, beam, vocab = lprobs.shape
    nrows = bsz * beam
    lp2 = lprobs.reshape(nrows, vocab)
    last = prev_output_tokens[:, -1].astype(jnp.int32)
    sc_sel = jnp.take(scores, step - 1, axis=2).reshape(nrows).astype(jnp.float32)
    bidx = original_batch_idxs.astype(jnp.int32)

    info = plsc.get_sparse_core_info()
    nc, ns = info.num_cores, info.num_subcores
    assert nc * ns == bsz, (nc, ns, bsz)

    mesh = plsc.VectorSubcoreMesh(core_axis_name="c", subcore_axis_name="s")
    f = pl.kernel(
        functools.partial(_sc_body, vocab, nc, beam),
        out_type=(
            jax.ShapeDtypeStruct((bsz * 16,), jnp.float32),
            jax.ShapeDtypeStruct((bsz * 16,), jnp.int32),
            jax.ShapeDtypeStruct((bsz * 16,), jnp.int32),
        ),
        mesh=mesh,
        compiler_params=pltpu.CompilerParams(needs_layout_passes=False),
        scratch_types=(
            pltpu.VMEM((bsz + 16,), jnp.int32),
            pltpu.VMEM((nrows + 16,), jnp.int32),
            pltpu.VMEM((nrows + 16,), jnp.float32),
            pltpu.VMEM((beam + 1, beam, _WINW), jnp.float32),
            pltpu.VMEM((beam * _ROW_PAD,), jnp.float32),
            pltpu.VMEM((beam * _ROW_PAD,), jnp.int32),
            pltpu.VMEM((16,), jnp.float32),
            pltpu.VMEM((16,), jnp.int32),
            pltpu.VMEM((16,), jnp.int32),
            pltpu.SemaphoreType.DMA,
        ),
    )
    s_out, t_out, b_out = f(lp2, last, sc_sel, bidx)
    return (s_out.reshape(bsz, 16)[:, :beam],
            t_out.reshape(bsz, 16)[:, :beam],
            b_out.reshape(bsz, 16)[:, :beam])


# speedup vs baseline: 80.5530x; 1.0593x over previous
import functools

import jax
import jax.numpy as jnp
from jax import lax
from jax.experimental import pallas as pl
from jax.experimental.pallas import tpu as pltpu
from jax.experimental.pallas import tpu_sc as plsc

_MULT = 977
_NALLOW = 100
_ROW_PAD = 112
_WINW = 256
_NEG = -3.4028235e38
_IMAX = 2**31 - 1
_NPOS = 1024


def _sc_body(vocab, nc, beam, lp_ref, prev_ref, sc_ref, bidx_ref, step_ref,
             scores_out, tokens_out, beams_out,
             bidx_v, prev_v, scb_v, step_v, win_v, vals_v, cidx_v,
             ov_f, ot_i, ob_i, sem):
    ncc = beam * _ROW_PAD // 16
    npad = _NPOS // 16
    minor_pad = 128
    w = lax.axis_index("s") * nc + lax.axis_index("c")
    lane = jnp.arange(16, dtype=jnp.int32)
    rows0 = pl.multiple_of(w * beam, 8)

    for h in [pltpu.async_copy(bidx_ref, bidx_v.at[pl.ds(0, bidx_ref.shape[0])], sem),
              pltpu.async_copy(prev_ref.at[pl.ds(rows0, beam), pl.ds(0, minor_pad)],
                               prev_v, sem),
              pltpu.async_copy(sc_ref.at[pl.ds(rows0, beam), pl.ds(0, minor_pad)],
                               scb_v, sem),
              pltpu.async_copy(step_ref, step_v, sem)]:
        h.wait()
    stepm1 = step_v[...][0] - 1
    colm1 = jnp.full((16,), stepm1, jnp.int32)
    b_id = bidx_v[pl.ds(w, 16)][0]

    def row_base(m):
        rowv = jnp.full((16,), m, jnp.int32)
        last_m = plsc.load_gather(prev_v, [rowv, colm1])[0]
        base = lax.rem(b_id * _MULT + last_m, vocab)
        s1 = lax.min(base - lax.rem(base, 128), ((vocab + 127) // 128) * 128 - _WINW)
        return base, s1

    handles = [pltpu.async_copy(
        lp_ref.at[pl.ds(rows0, beam), pl.ds(0, _WINW)], win_v.at[beam], sem)]
    for m in range(beam):
        _, s1 = row_base(m)
        handles.append(pltpu.async_copy(
            lp_ref.at[pl.ds(rows0, beam), pl.ds(pl.multiple_of(s1, 128), _WINW)],
            win_v.at[m], sem))
    for h in handles:
        h.wait()

    negs = jnp.full((16,), _NEG, jnp.float32)
    imaxs = jnp.full((16,), _IMAX, jnp.int32)

    def gather_row(m, _):
        rowv = jnp.full((16,), m, jnp.int32)
        base, s1 = row_base(m)
        sc_m = plsc.load_gather(scb_v, [rowv, colm1])[0]
        for c in range(_ROW_PAD // 16):
            j = lane + 16 * c
            idv = base + j
            wrapped = idv >= vocab
            tok = jnp.where(wrapped, idv - vocab, idv)
            sel = jnp.where(wrapped, beam, m)
            off = jnp.where(wrapped, tok, idv - s1)
            valid = j < _NALLOW
            off = jnp.where(valid, off, 0)
            g = plsc.load_gather(win_v, [sel, rowv, off])
            vals_v[pl.ds(m * _ROW_PAD + 16 * c, 16)] = jnp.where(valid, g + sc_m, _NEG)
            cidx_v[pl.ds(m * _ROW_PAD + 16 * c, 16)] = jnp.where(valid, m * vocab + tok, _IMAX)
        return 0

    lax.fori_loop(0, beam, gather_row, 0)

    def pad_chunk(cc, _):
        vals_v[pl.ds(cc * 16, 16)] = negs
        cidx_v[pl.ds(cc * 16, 16)] = imaxs
        return 0

    lax.fori_loop(ncc, npad, pad_chunk, 0)

    def lex_merge(av, ai, ap, v, i, p):
        upd = (v > av) | ((v == av) & (i < ai))
        return (jnp.where(upd, v, av), jnp.where(upd, i, ai), jnp.where(upd, p, ap))

    def scan4(ci, carry):
        bv, bi, bp = carry
        for k in range(4):
            off = (ci * 4 + k) * 16
            v = vals_v[pl.ds(off, 16)]
            i = cidx_v[pl.ds(off, 16)]
            bv, bi, bp = lex_merge(bv, bi, bp, v, i, off + lane)
        return bv, bi, bp

    bests = lax.fori_loop(0, npad // 4, scan4, (negs, imaxs, imaxs))

    def round_body(rnd, carry):
        bests_v, bests_i, bests_p, out_val, out_idx = carry
        mval = jnp.max(bests_v)
        wi = jnp.min(jnp.where(bests_v == mval, bests_i, _IMAX))
        wp = jnp.min(jnp.where((bests_v == mval) & (bests_i == wi), bests_p, _IMAX))
        out_val = jnp.where(lane == rnd, mval, out_val)
        out_idx = jnp.where(lane == rnd, wi, out_idx)
        plsc.store_scatter(vals_v, [jnp.full((16,), wp, jnp.int32)],
                           negs, mask=lane == 0)
        col = lax.rem(wp, 16)
        cv, ci_, cp = negs, imaxs, imaxs
        for k in range(4):
            pos = col + 16 * (k * 16 + lane)
            v = plsc.load_gather(vals_v, [pos])
            i = plsc.load_gather(cidx_v, [pos])
            cv, ci_, cp = lex_merge(cv, ci_, cp, v, i, pos)
        nm = jnp.max(cv)
        ni = jnp.min(jnp.where(cv == nm, ci_, _IMAX))
        np_ = jnp.min(jnp.where((cv == nm) & (ci_ == ni), cp, _IMAX))
        cl = lane == col
        return (jnp.where(cl, nm, bests_v), jnp.where(cl, ni, bests_i),
                jnp.where(cl, np_, bests_p), out_val, out_idx)

    _, _, _, out_val, out_idx = lax.fori_loop(
        0, beam, round_body, (bests[0], bests[1], bests[2], negs, imaxs))

    beams = lax.div(out_idx, jnp.int32(vocab))
    toks = out_idx - beams * vocab
    ov_f[...] = out_val
    ot_i[...] = toks
    ob_i[...] = beams
    dst = pl.ds(pl.multiple_of(w * beam, 8), beam)
    src = pl.ds(0, beam)
    for h in [pltpu.async_copy(ov_f.at[src], scores_out.at[dst], sem),
              pltpu.async_copy(ot_i.at[src], tokens_out.at[dst], sem),
              pltpu.async_copy(ob_i.at[src], beams_out.at[dst], sem)]:
        h.wait()


def kernel(step, lprobs, scores, prev_output_tokens, original_batch_idxs, prefix_mention_is):
    bsz, beam, vocab = lprobs.shape
    nrows = bsz * beam
    lp2 = lprobs.reshape(nrows, vocab)
    prev = prev_output_tokens.astype(jnp.int32)
    sc2 = scores.reshape(nrows, scores.shape[2]).astype(jnp.float32)
    bidx = original_batch_idxs.astype(jnp.int32)
    stepv = jnp.full((16,), step, jnp.int32)

    info = plsc.get_sparse_core_info()
    nc, ns = info.num_cores, info.num_subcores
    assert nc * ns == bsz, (nc, ns, bsz)

    mesh = plsc.VectorSubcoreMesh(core_axis_name="c", subcore_axis_name="s")
    f = pl.kernel(
        functools.partial(_sc_body, vocab, nc, beam),
        out_type=(
            jax.ShapeDtypeStruct((nrows,), jnp.float32),
            jax.ShapeDtypeStruct((nrows,), jnp.int32),
            jax.ShapeDtypeStruct((nrows,), jnp.int32),
        ),
        mesh=mesh,
        compiler_params=pltpu.CompilerParams(needs_layout_passes=False),
        scratch_types=(
            pltpu.VMEM((bsz + 16,), jnp.int32),
            pltpu.VMEM((beam, 128), jnp.int32),
            pltpu.VMEM((beam, 128), jnp.float32),
            pltpu.VMEM((16,), jnp.int32),
            pltpu.VMEM((beam + 1, beam, _WINW), jnp.float32),
            pltpu.VMEM((_NPOS,), jnp.float32),
            pltpu.VMEM((_NPOS,), jnp.int32),
            pltpu.VMEM((16,), jnp.float32),
            pltpu.VMEM((16,), jnp.int32),
            pltpu.VMEM((16,), jnp.int32),
            pltpu.SemaphoreType.DMA,
        ),
    )
    s_out, t_out, b_out = f(lp2, prev, sc2, bidx, stepv)
    return (s_out.reshape(bsz, beam),
            t_out.reshape(bsz, beam),
            b_out.reshape(bsz, beam))
